# baseline (speedup 1.0000x reference)
import functools

import jax
import jax.numpy as jnp
from jax import lax
from jax.experimental import pallas as pl
from jax.experimental.pallas import tpu as pltpu
from jax.experimental.pallas import tpu_sc as plsc

EMBED_DIM = 1024
HALF_DIM = 512
K_NEIGHBORS = 32
B, S = 2, 2048
RB = 256
SR = 64
NROWS = B * S
NC, NS, L = 2, 16, 16
NW = NC * NS
ROWS_PER_W = S // NW
GROUPS_PER_W = ROWS_PER_W // L
CHUNK = 32
NCHUNK = S // CHUNK
SPAD = S + 8


def _bf16_dot_nt(a, b):
    return jax.lax.dot_general(
        a.astype(jnp.bfloat16), b.astype(jnp.bfloat16),
        (((1,), (1,)), ((), ())), preferred_element_type=jnp.float32)


def _proj_kernel(e_ref, wp_ref, bp_ref, out_ref):
    out_ref[0] = _bf16_dot_nt(e_ref[0], wp_ref[...]) + bp_ref[...]


def _dist_kernel(proj_ref, prow_ref, out_ref):
    proj = proj_ref[0]
    ones = jnp.ones((1, HALF_DIM), jnp.float32)
    sqc = jax.lax.dot_general(
        ones, proj * proj, (((1,), (1,)), ((), ())),
        preferred_element_type=jnp.float32,
        precision=jax.lax.Precision.HIGHEST)
    proj_bf = proj.astype(jnp.bfloat16)

    def strip(s, _):
        prow = prow_ref[0, pl.ds(s * SR, SR), :]
        inner = jax.lax.dot_general(
            prow.astype(jnp.bfloat16), proj_bf,
            (((1,), (1,)), ((), ())),
            preferred_element_type=jnp.float32)
        sqr = jnp.sum(prow * prow, axis=1, keepdims=True)
        d2 = (sqr + sqc) - 2.0 * inner
        out_ref[pl.ds(s * SR, SR), :] = jnp.sqrt(jnp.maximum(d2, 0.0))
        return 0

    jax.lax.fori_loop(0, RB // SR, strip, 0)


def _select_kernel(dist_hbm, zero_hbm, mask_hbm, blk, mblk, ilist, sem):
    wid = lax.axis_index("s") * NC + lax.axis_index("c")
    lanebase = jnp.arange(L, dtype=jnp.int32) * SPAD
    lanemask = jnp.arange(L, dtype=jnp.int32) * S
    inf = jnp.full((L,), jnp.inf, jnp.float32)
    ones_f = jnp.ones((L,), jnp.float32)
    zeros_f = jnp.zeros((L,), jnp.float32)

    pltpu.sync_copy(zero_hbm, mblk)

    def group(g, _):
        base = (wid * ROWS_PER_W + g * L) * S
        cps = [pltpu.async_copy(dist_hbm.at[pl.ds(base + l * S, S)],
                                blk.at[pl.ds(l * SPAD, S)], sem)
               for l in range(L)]
        for cp in cps:
            cp.wait()

        def chunk_scan(c, bs):
            a0 = lanebase + c * CHUNK
            m = None
            for k0 in range(0, CHUNK, 16):
                vs = [plsc.load_gather(blk, [a0 + (k0 + k)])
                      for k in range(16)]
                while len(vs) > 1:
                    vs = [jnp.minimum(vs[i], vs[i + 1])
                          for i in range(0, len(vs), 2)]
                m = vs[0] if m is None else jnp.minimum(m, vs[0])
            out = []
            for bi in bs:
                out.append(jnp.minimum(bi, m))
                m = jnp.maximum(bi, m)
            return tuple(out)

        p1buf = lax.fori_loop(0, NCHUNK, chunk_scan,
                              tuple(inf for _ in range(K_NEIGHBORS)))
        that = p1buf[K_NEIGHBORS - 1]

        def collect(c, cnt):
            for k in range(CHUNK):
                j = c * CHUNK + k
                v = plsc.load_gather(blk, [lanebase + j])
                msk = v <= that
                plsc.store_scatter(ilist, [lanebase + cnt],
                                   jnp.full((L,), j, jnp.int32), mask=msk)
                cnt = cnt + jnp.where(msk, 1, 0).astype(jnp.int32)
            return cnt

        cnt = lax.fori_loop(0, NCHUNK, collect,
                            jnp.zeros((L,), jnp.int32))
        maxc = jnp.max(cnt)

        def cand(i):
            valid = i < cnt
            jv = plsc.load_gather(ilist, [lanebase + i])
            jv = jnp.where(valid, jv, 0)
            v = plsc.load_gather(blk, [lanebase + jv])
            return valid, jv, jnp.where(valid, v, jnp.inf)

        def ins(i, bs):
            _, _, v = cand(i)
            c = v
            out = []
            for bi in bs:
                out.append(jnp.minimum(bi, c))
                c = jnp.maximum(bi, c)
            return tuple(out)

        buf = lax.fori_loop(0, maxc, ins,
                            tuple(inf for _ in range(K_NEIGHBORS)))
        t = buf[K_NEIGHBORS - 1]
        n_lt = jnp.zeros((L,), jnp.int32)
        for bi in buf:
            n_lt = n_lt + jnp.where(bi < t, 1, 0).astype(jnp.int32)
        m_need = K_NEIGHBORS - n_lt

        def emit(i, seen_eq):
            valid, jv, v = cand(i)
            is_eq = v == t
            sel = ((v < t) | (is_eq & (seen_eq < m_need))) & valid
            plsc.store_scatter(mblk, [lanemask + jv], ones_f, mask=sel)
            return seen_eq + jnp.where(is_eq, 1, 0).astype(jnp.int32)

        lax.fori_loop(0, maxc, emit, jnp.zeros((L,), jnp.int32))
        pltpu.sync_copy(mblk, mask_hbm.at[pl.ds(base, L * S)])

        def rezero(i, _):
            valid = i < cnt
            jv = plsc.load_gather(ilist, [lanebase + i])
            jv = jnp.where(valid, jv, 0)
            plsc.store_scatter(mblk, [lanemask + jv], zeros_f, mask=valid)
            return 0

        lax.fori_loop(0, maxc, rezero, 0)
        return 0

    lax.fori_loop(0, GROUPS_PER_W, group, 0)


def _pool_kernel(mask_ref, ehh_ref, ehl_ref, we_ref, be_ref, g_ref, b_ref,
                 out_ref):
    def strip(s, _):
        m = mask_ref[pl.ds(s * SR, SR), :].astype(jnp.bfloat16)
        nn = (((1,), (0,)), ((), ()))
        pooled = (jax.lax.dot_general(
            m, ehh_ref[0], nn, preferred_element_type=jnp.float32)
            + jax.lax.dot_general(
                m, ehl_ref[0], nn, preferred_element_type=jnp.float32)
        ) * (1.0 / K_NEIGHBORS)
        x = _bf16_dot_nt(pooled, we_ref[...]) + be_ref[...]
        mu = jnp.mean(x, axis=1, keepdims=True)
        xc = x - mu
        var = jnp.mean(xc * xc, axis=1, keepdims=True)
        x = xc / jnp.sqrt(var + 1e-5)
        x = x * g_ref[...] + b_ref[...]
        out_ref[pl.ds(s * SR, SR), :] = jnp.maximum(x, 0.0)
        return 0

    jax.lax.fori_loop(0, RB // SR, strip, 0)


@jax.jit
def kernel(embeddings, Wp, bp, We, be, gamma, beta):
    d, h = EMBED_DIM, HALF_DIM
    bp2 = bp.reshape(1, h)
    be2 = be.reshape(1, d)
    g2 = gamma.reshape(1, d)
    b2 = beta.reshape(1, d)

    proj = pl.pallas_call(
        _proj_kernel,
        grid=(B, S // RB),
        in_specs=[
            pl.BlockSpec((1, RB, d), lambda b, i: (b, i, 0)),
            pl.BlockSpec((h, d), lambda b, i: (0, 0)),
            pl.BlockSpec((1, h), lambda b, i: (0, 0)),
        ],
        out_specs=pl.BlockSpec((1, RB, h), lambda b, i: (b, i, 0)),
        out_shape=jax.ShapeDtypeStruct((B, S, h), jnp.float32),
    )(embeddings, Wp, bp2)

    nblk = S // RB

    def dist_batch(batch):
        return pl.pallas_call(
            _dist_kernel,
            grid=(nblk,),
            in_specs=[
                pl.BlockSpec((1, S, h), lambda i: (batch, 0, 0)),
                pl.BlockSpec((1, RB, h), lambda i: (batch, i, 0)),
            ],
            out_specs=pl.BlockSpec((RB, S), lambda i: (i, 0)),
            out_shape=jax.ShapeDtypeStruct((S, S), jnp.float32),
        )(proj, proj)

    sel = functools.partial(
        pl.kernel,
        out_type=jax.ShapeDtypeStruct((S * S,), jnp.float32),
        mesh=plsc.VectorSubcoreMesh(core_axis_name="c", subcore_axis_name="s"),
        scratch_types=[
            pltpu.VMEM((L * SPAD,), jnp.float32),
            pltpu.VMEM((L * S,), jnp.float32),
            pltpu.VMEM((L * SPAD,), jnp.int32),
            pltpu.SemaphoreType.DMA,
        ],
        compiler_params=pltpu.CompilerParams(needs_layout_passes=False),
    )(_select_kernel)
    zeros_blk = jnp.zeros((L * S,), jnp.float32)
    d0 = dist_batch(0)
    m0 = sel(d0.reshape(S * S), zeros_blk)
    d1 = dist_batch(1)
    m1 = sel(d1.reshape(S * S), zeros_blk)

    eh = embeddings[:, :, :h]
    eh_hi = eh.astype(jnp.bfloat16)
    eh_lo = (eh - eh_hi.astype(jnp.float32)).astype(jnp.bfloat16)

    def pool(mask_b, batch):
        return pl.pallas_call(
            _pool_kernel,
            grid=(nblk,),
            in_specs=[
                pl.BlockSpec((RB, S), lambda i: (i, 0)),
                pl.BlockSpec((1, S, h), lambda i: (batch, 0, 0)),
                pl.BlockSpec((1, S, h), lambda i: (batch, 0, 0)),
                pl.BlockSpec((d, h), lambda i: (0, 0)),
                pl.BlockSpec((1, d), lambda i: (0, 0)),
                pl.BlockSpec((1, d), lambda i: (0, 0)),
                pl.BlockSpec((1, d), lambda i: (0, 0)),
            ],
            out_specs=pl.BlockSpec((RB, d), lambda i: (i, 0)),
            out_shape=jax.ShapeDtypeStruct((S, d), jnp.float32),
        )(mask_b.reshape(S, S), eh_hi, eh_lo, We, be2, g2, b2)

    o0 = pool(m0, 0)
    o1 = pool(m1, 1)
    return jnp.stack([o0, o1])


# speedup vs baseline: 1.0274x; 1.0274x over previous
import functools

import jax
import jax.numpy as jnp
from jax import lax
from jax.experimental import pallas as pl
from jax.experimental.pallas import tpu as pltpu
from jax.experimental.pallas import tpu_sc as plsc

EMBED_DIM = 1024
HALF_DIM = 512
K_NEIGHBORS = 32
B, S = 2, 2048
RB = 256
SR = 64
NROWS = B * S
NC, NS, L = 2, 16, 16
NW = NC * NS
ROWS_PER_W = S // NW
GROUPS_PER_W = ROWS_PER_W // L
CHUNK = 32
NCHUNK = S // CHUNK
SPAD = S + 8


def _bf16_dot_nt(a, b):
    return jax.lax.dot_general(
        a.astype(jnp.bfloat16), b.astype(jnp.bfloat16),
        (((1,), (1,)), ((), ())), preferred_element_type=jnp.float32)


def _proj_kernel(e_ref, wp_ref, bp_ref, out_ref):
    out_ref[0] = _bf16_dot_nt(e_ref[0], wp_ref[...]) + bp_ref[...]


def _dist_kernel(proj_ref, prow_ref, out_ref):
    proj = proj_ref[0]
    ones = jnp.ones((1, HALF_DIM), jnp.float32)
    sqc = jax.lax.dot_general(
        ones, proj * proj, (((1,), (1,)), ((), ())),
        preferred_element_type=jnp.float32,
        precision=jax.lax.Precision.HIGHEST)
    proj_bf = proj.astype(jnp.bfloat16)

    def strip(s, _):
        prow = prow_ref[0, pl.ds(s * SR, SR), :]
        inner = jax.lax.dot_general(
            prow.astype(jnp.bfloat16), proj_bf,
            (((1,), (1,)), ((), ())),
            preferred_element_type=jnp.float32)
        sqr = jnp.sum(prow * prow, axis=1, keepdims=True)
        d2 = (sqr + sqc) - 2.0 * inner
        out_ref[pl.ds(s * SR, SR), :] = jnp.sqrt(jnp.maximum(d2, 0.0))
        return 0

    jax.lax.fori_loop(0, RB // SR, strip, 0)


def _select_kernel(dist_hbm, zero_hbm, mask_hbm, blk, mblk, ilist, sem):
    wid = lax.axis_index("s") * NC + lax.axis_index("c")
    lanebase = jnp.arange(L, dtype=jnp.int32) * SPAD
    lanemask = jnp.arange(L, dtype=jnp.int32) * S
    inf = jnp.full((L,), jnp.inf, jnp.float32)
    ones_f = jnp.ones((L,), jnp.float32)
    zeros_f = jnp.zeros((L,), jnp.float32)

    pltpu.sync_copy(zero_hbm, mblk)

    def group(g, _):
        base = (wid * ROWS_PER_W + g * L) * S
        cps = [pltpu.async_copy(dist_hbm.at[pl.ds(base + l * S, S)],
                                blk.at[pl.ds(l * SPAD, S)], sem)
               for l in range(L)]
        for cp in cps:
            cp.wait()

        def chunk_scan(c, bs):
            a0 = lanebase + c * CHUNK
            m = None
            for k0 in range(0, CHUNK, 16):
                vs = [plsc.load_gather(blk, [a0 + (k0 + k)])
                      for k in range(16)]
                while len(vs) > 1:
                    vs = [jnp.minimum(vs[i], vs[i + 1])
                          for i in range(0, len(vs), 2)]
                m = vs[0] if m is None else jnp.minimum(m, vs[0])
            out = []
            for bi in bs:
                out.append(jnp.minimum(bi, m))
                m = jnp.maximum(bi, m)
            return tuple(out)

        p1buf = lax.fori_loop(0, NCHUNK, chunk_scan,
                              tuple(inf for _ in range(K_NEIGHBORS)))
        that = p1buf[K_NEIGHBORS - 1]

        def collect(c, cnt):
            for k in range(CHUNK):
                j = c * CHUNK + k
                v = plsc.load_gather(blk, [lanebase + j])
                msk = v <= that
                plsc.store_scatter(ilist, [lanebase + cnt],
                                   jnp.full((L,), j, jnp.int32), mask=msk)
                cnt = cnt + jnp.where(msk, 1, 0).astype(jnp.int32)
            return cnt

        cnt = lax.fori_loop(0, NCHUNK, collect,
                            jnp.zeros((L,), jnp.int32))
        maxc = jnp.max(cnt)

        def cand(i):
            valid = i < cnt
            jv = plsc.load_gather(ilist, [lanebase + i])
            jv = jnp.where(valid, jv, 0)
            v = plsc.load_gather(blk, [lanebase + jv])
            return valid, jv, jnp.where(valid, v, jnp.inf)

        def ins(i, bs):
            _, _, v = cand(i)
            c = v
            out = []
            for bi in bs:
                out.append(jnp.minimum(bi, c))
                c = jnp.maximum(bi, c)
            return tuple(out)

        buf = lax.fori_loop(0, maxc, ins,
                            tuple(inf for _ in range(K_NEIGHBORS)))
        t = buf[K_NEIGHBORS - 1]
        n_lt = jnp.zeros((L,), jnp.int32)
        for bi in buf:
            n_lt = n_lt + jnp.where(bi < t, 1, 0).astype(jnp.int32)
        m_need = K_NEIGHBORS - n_lt

        def emit(i, seen_eq):
            valid, jv, v = cand(i)
            is_eq = v == t
            sel = ((v < t) | (is_eq & (seen_eq < m_need))) & valid
            plsc.store_scatter(mblk, [lanemask + jv], ones_f, mask=sel)
            return seen_eq + jnp.where(is_eq, 1, 0).astype(jnp.int32)

        lax.fori_loop(0, maxc, emit, jnp.zeros((L,), jnp.int32))
        pltpu.sync_copy(mblk, mask_hbm.at[pl.ds(base, L * S)])

        def rezero(i, _):
            valid = i < cnt
            jv = plsc.load_gather(ilist, [lanebase + i])
            jv = jnp.where(valid, jv, 0)
            plsc.store_scatter(mblk, [lanemask + jv], zeros_f, mask=valid)
            return 0

        lax.fori_loop(0, maxc, rezero, 0)
        return 0

    lax.fori_loop(0, GROUPS_PER_W, group, 0)


def _pool_kernel(mask_ref, ehh_ref, we_ref, be_ref, g_ref, b_ref, out_ref):
    def strip(s, _):
        m = mask_ref[pl.ds(s * SR, SR), :].astype(jnp.bfloat16)
        pooled = jax.lax.dot_general(
            m, ehh_ref[0], (((1,), (0,)), ((), ())),
            preferred_element_type=jnp.float32) * (1.0 / K_NEIGHBORS)
        x = _bf16_dot_nt(pooled, we_ref[...]) + be_ref[...]
        mu = jnp.mean(x, axis=1, keepdims=True)
        xc = x - mu
        var = jnp.mean(xc * xc, axis=1, keepdims=True)
        x = xc / jnp.sqrt(var + 1e-5)
        x = x * g_ref[...] + b_ref[...]
        out_ref[pl.ds(s * SR, SR), :] = jnp.maximum(x, 0.0)
        return 0

    jax.lax.fori_loop(0, RB // SR, strip, 0)


@jax.jit
def kernel(embeddings, Wp, bp, We, be, gamma, beta):
    d, h = EMBED_DIM, HALF_DIM
    bp2 = bp.reshape(1, h)
    be2 = be.reshape(1, d)
    g2 = gamma.reshape(1, d)
    b2 = beta.reshape(1, d)

    proj = pl.pallas_call(
        _proj_kernel,
        grid=(B, S // RB),
        in_specs=[
            pl.BlockSpec((1, RB, d), lambda b, i: (b, i, 0)),
            pl.BlockSpec((h, d), lambda b, i: (0, 0)),
            pl.BlockSpec((1, h), lambda b, i: (0, 0)),
        ],
        out_specs=pl.BlockSpec((1, RB, h), lambda b, i: (b, i, 0)),
        out_shape=jax.ShapeDtypeStruct((B, S, h), jnp.float32),
    )(embeddings, Wp, bp2)

    nblk = S // RB

    def dist_batch(batch):
        return pl.pallas_call(
            _dist_kernel,
            grid=(nblk,),
            in_specs=[
                pl.BlockSpec((1, S, h), lambda i: (batch, 0, 0)),
                pl.BlockSpec((1, RB, h), lambda i: (batch, i, 0)),
            ],
            out_specs=pl.BlockSpec((RB, S), lambda i: (i, 0)),
            out_shape=jax.ShapeDtypeStruct((S, S), jnp.float32),
        )(proj, proj)

    sel = functools.partial(
        pl.kernel,
        out_type=jax.ShapeDtypeStruct((S * S,), jnp.float32),
        mesh=plsc.VectorSubcoreMesh(core_axis_name="c", subcore_axis_name="s"),
        scratch_types=[
            pltpu.VMEM((L * SPAD,), jnp.float32),
            pltpu.VMEM((L * S,), jnp.float32),
            pltpu.VMEM((L * SPAD,), jnp.int32),
            pltpu.SemaphoreType.DMA,
        ],
        compiler_params=pltpu.CompilerParams(needs_layout_passes=False),
    )(_select_kernel)
    zeros_blk = jnp.zeros((L * S,), jnp.float32)
    d0 = dist_batch(0)
    m0 = sel(d0.reshape(S * S), zeros_blk)
    d1 = dist_batch(1)
    m1 = sel(d1.reshape(S * S), zeros_blk)

    eh_hi = embeddings[:, :, :h].astype(jnp.bfloat16)

    def pool(mask_b, batch):
        return pl.pallas_call(
            _pool_kernel,
            grid=(nblk,),
            in_specs=[
                pl.BlockSpec((RB, S), lambda i: (i, 0)),
                pl.BlockSpec((1, S, h), lambda i: (batch, 0, 0)),
                pl.BlockSpec((d, h), lambda i: (0, 0)),
                pl.BlockSpec((1, d), lambda i: (0, 0)),
                pl.BlockSpec((1, d), lambda i: (0, 0)),
                pl.BlockSpec((1, d), lambda i: (0, 0)),
            ],
            out_specs=pl.BlockSpec((RB, d), lambda i: (i, 0)),
            out_shape=jax.ShapeDtypeStruct((S, d), jnp.float32),
        )(mask_b.reshape(S, S), eh_hi, We, be2, g2, b2)

    o0 = pool(m0, 0)
    o1 = pool(m1, 1)
    return jnp.stack([o0, o1])
